# baseline (speedup 1.0000x reference)
import functools

import jax
import jax.numpy as jnp
from jax import lax
from jax.experimental import pallas as pl
from jax.experimental.pallas import tpu as pltpu
from jax.experimental.pallas import tpu_sc as plsc

D_MODEL = 64

_NC = 2
_NS = 16
_NW = _NC * _NS
_CH = 40
_NB = 6
_GA = 3


def _pair_gather(table, idx_flat):
    B = idx_flat.shape[0]
    b_per_w = B // _NW
    h_per_w = b_per_w // 2
    q_per_w = b_per_w // 400
    n_chunks = h_per_w // _CH
    mpq = 200 // _CH
    mesh = plsc.VectorSubcoreMesh(core_axis_name="c", subcore_axis_name="s")

    @functools.partial(
        pl.kernel,
        mesh=mesh,
        compiler_params=pltpu.CompilerParams(
            use_tc_tiling_on_sc=False, needs_layout_passes=False
        ),
        out_type=jax.ShapeDtypeStruct((B // 400, 200, 2 * D_MODEL),
                                      jnp.float32),
        scratch_types=[
            pltpu.VMEM((b_per_w,), jnp.int32),
            pltpu.VMEM((h_per_w,), jnp.int32),
            pltpu.VMEM((h_per_w,), jnp.int32),
            pltpu.VMEM((_NB, _CH, D_MODEL), jnp.float32),
            pltpu.VMEM((_NB, _CH, D_MODEL), jnp.float32),
            pltpu.SemaphoreType.DMA((_NB,)),
            pltpu.SemaphoreType.DMA((_NB,)),
            pltpu.SemaphoreType.DMA((_NB,)),
            pltpu.SemaphoreType.DMA((_NB,)),
        ],
    )
    def k(table_hbm, idx_hbm, out_hbm,
          idx_v, idxe_v, idxo_v, rowse_v, rowso_v,
          gseme, gsemo, wseme, wsemo):
        wid = lax.axis_index("s") * _NC + lax.axis_index("c")
        qbase = wid * q_per_w
        pltpu.sync_copy(idx_hbm.at[pl.ds(wid * b_per_w, b_per_w)], idx_v)

        lanes = lax.iota(jnp.int32, 16)

        def deint(g, carry):
            pos = lanes * 2 + g * 32
            idxe_v[pl.ds(g * 16, 16)] = plsc.load_gather(idx_v, [pos])
            idxo_v[pl.ds(g * 16, 16)] = plsc.load_gather(idx_v, [pos + 1])
            return carry

        lax.fori_loop(0, h_per_w // 16, deint, 0)

        def issue_gather(chunk, buf):
            pltpu.async_copy(
                table_hbm.at[idxe_v.at[pl.ds(chunk * _CH, _CH)]],
                rowse_v.at[buf],
                gseme.at[buf],
            )
            pltpu.async_copy(
                table_hbm.at[idxo_v.at[pl.ds(chunk * _CH, _CH)]],
                rowso_v.at[buf],
                gsemo.at[buf],
            )

        def wait_gather(buf):
            pltpu.make_async_copy(
                table_hbm.at[pl.ds(0, _CH)], rowse_v.at[buf], gseme.at[buf]
            ).wait()
            pltpu.make_async_copy(
                table_hbm.at[pl.ds(0, _CH)], rowso_v.at[buf], gsemo.at[buf]
            ).wait()

        def issue_write(chunk, buf):
            q = qbase + chunk // mpq
            m0 = (chunk % mpq) * _CH
            pltpu.async_copy(
                rowse_v.at[buf],
                out_hbm.at[q, pl.ds(m0, _CH), pl.ds(0, D_MODEL)],
                wseme.at[buf],
            )
            pltpu.async_copy(
                rowso_v.at[buf],
                out_hbm.at[q, pl.ds(m0, _CH), pl.ds(D_MODEL, D_MODEL)],
                wsemo.at[buf],
            )

        def wait_write(buf):
            pltpu.make_async_copy(
                rowse_v.at[buf],
                out_hbm.at[qbase, pl.ds(0, _CH), pl.ds(0, D_MODEL)],
                wseme.at[buf],
            ).wait()
            pltpu.make_async_copy(
                rowso_v.at[buf],
                out_hbm.at[qbase, pl.ds(0, _CH), pl.ds(D_MODEL, D_MODEL)],
                wsemo.at[buf],
            ).wait()

        for j in range(_GA):
            issue_gather(j, j)

        def body(i, carry):
            b = lax.rem(i, _NB)
            wait_gather(b)
            issue_write(i, b)
            nxt = i + _GA

            @pl.when(nxt < n_chunks)
            def _():
                bn = lax.rem(nxt, _NB)

                @pl.when(nxt >= _NB)
                def _():
                    wait_write(bn)

                issue_gather(nxt, bn)

            return carry

        lax.fori_loop(0, n_chunks, body, 0)

        for j in range(_NB):
            wait_write(j)

    return k(table, idx_flat)


def _retile(z):
    def body(z_ref, y_ref):
        def step(m2, carry):
            a = z_ref[:, m2, :]
            b = z_ref[:, m2 + 100, :]
            for p in (0, 1):
                at = a[:, p * 64:(p + 1) * 64].T
                bt = b[:, p * 64:(p + 1) * 64].T
                y_ref[2 * m2 + p] = jnp.stack(
                    [at, bt], axis=-1).reshape(64, 128)
            return carry

        lax.fori_loop(0, 100, step, 0)

    return pl.pallas_call(
        body,
        grid=(32,),
        in_specs=[pl.BlockSpec((64, 200, 128), lambda it: (it, 0, 0))],
        out_specs=pl.BlockSpec((200, 64, 128), lambda it: (0, 0, it)),
        out_shape=jax.ShapeDtypeStruct((200, 64, 4096), jnp.float32),
    )(z)


def kernel(indices, pe):
    b0, b1 = indices.shape
    flat = indices.reshape(b0 * b1).astype(jnp.int32)
    z = _pair_gather(pe, flat)
    y = _retile(z)
    return jnp.transpose(y, (2, 0, 1))


# speedup vs baseline: 31.0949x; 31.0949x over previous
import functools

import jax
import jax.numpy as jnp
from jax import lax
from jax.experimental import pallas as pl
from jax.experimental.pallas import tpu as pltpu
from jax.experimental.pallas import tpu_sc as plsc

D_MODEL = 64

_NC = 2
_NS = 16
_NW = _NC * _NS
_NB = 2
_GL = 104


def _pair_gather(table, idx_flat):
    B = idx_flat.shape[0]
    b_per_w = B // _NW
    h_per_w = b_per_w // 2
    n_pairs = 64
    mesh = plsc.VectorSubcoreMesh(core_axis_name="c", subcore_axis_name="s")

    @functools.partial(
        pl.kernel,
        mesh=mesh,
        compiler_params=pltpu.CompilerParams(
            use_tc_tiling_on_sc=False, needs_layout_passes=False
        ),
        out_type=jax.ShapeDtypeStruct((B // 400, 200, 2 * D_MODEL),
                                      jnp.float32),
        scratch_types=[
            pltpu.VMEM((b_per_w,), jnp.int32),
            pltpu.VMEM((h_per_w,), jnp.int32),
            pltpu.VMEM((h_per_w,), jnp.int32),
            pltpu.VMEM((_NB, 4, _GL, D_MODEL), jnp.float32),
            pltpu.SemaphoreType.DMA((_NB,)),
            pltpu.SemaphoreType.DMA((_NB,)),
        ],
    )
    def k(table_hbm, idx_hbm, out_hbm,
          idx_v, idxe_v, idxo_v, rows_v, gsem, wsem):
        wid = lax.axis_index("s") * _NC + lax.axis_index("c")
        qbase = wid * n_pairs
        pltpu.sync_copy(idx_hbm.at[pl.ds(wid * b_per_w, b_per_w)], idx_v)

        lanes = lax.iota(jnp.int32, 16)

        def deint(g, carry):
            pos = lanes * 2 + g * 32
            idxe_v[pl.ds(g * 16, 16)] = plsc.load_gather(idx_v, [pos])
            idxo_v[pl.ds(g * 16, 16)] = plsc.load_gather(idx_v, [pos + 1])
            return carry

        lax.fori_loop(0, h_per_w // 16, deint, 0)

        def issue_gather(pi, b):
            off0 = pi * 200
            off1 = pi * 200 + 96
            for s, (iv, off) in enumerate(
                    ((idxe_v, off0), (idxe_v, off1),
                     (idxo_v, off0), (idxo_v, off1))):
                pltpu.async_copy(
                    table_hbm.at[iv.at[pl.ds(off, _GL)]],
                    rows_v.at[b, s],
                    gsem.at[b],
                )

        def wait_gather(b):
            for _ in range(4):
                pltpu.make_async_copy(
                    table_hbm.at[pl.ds(0, _GL)], rows_v.at[b, 0], gsem.at[b]
                ).wait()

        def issue_write(pi, b):
            il0 = 2 * pi
            par = il0 // 64
            q0 = qbase + il0 - 64 * par
            mb = 100 * par
            for s, (q, r0, c0) in enumerate(
                    ((q0, 0, 0), (q0 + 1, 4, 0),
                     (q0, 0, D_MODEL), (q0 + 1, 4, D_MODEL))):
                pltpu.async_copy(
                    rows_v.at[b, s, pl.ds(r0, 100)],
                    out_hbm.at[q, pl.ds(mb, 100), pl.ds(c0, D_MODEL)],
                    wsem.at[b],
                )

        def wait_write(b):
            for _ in range(4):
                pltpu.make_async_copy(
                    rows_v.at[b, 0, pl.ds(0, 100)],
                    out_hbm.at[qbase, pl.ds(0, 100), pl.ds(0, D_MODEL)],
                    wsem.at[b],
                ).wait()

        issue_gather(0, 0)

        def body(i, carry):
            b = lax.rem(i, _NB)
            wait_gather(b)
            issue_write(i, b)
            nxt = i + 1

            @pl.when(nxt < n_pairs)
            def _():
                bn = lax.rem(nxt, _NB)

                @pl.when(nxt >= _NB)
                def _():
                    wait_write(bn)

                issue_gather(nxt, bn)

            return carry

        lax.fori_loop(0, n_pairs, body, 0)

        for j in range(_NB):
            wait_write(j)

    return k(table, idx_flat)


def _retile(z):
    def body(z_ref, y_ref):
        eye = jax.lax.broadcasted_iota(jnp.int32, (64, 64), 0)
        eye = (eye == jax.lax.broadcasted_iota(jnp.int32, (64, 64), 1))
        eye = eye.astype(jnp.float32)
        zt = jax.lax.dot_general(
            z_ref[...].reshape(64, 200 * 128), eye,
            dimension_numbers=(((0,), (0,)), ((), ())),
            preferred_element_type=jnp.float32,
        )
        ya = zt[:100 * 128].reshape(200, 64, 64)
        yb = zt[100 * 128:].reshape(200, 64, 64)
        y_ref[...] = jnp.concatenate([ya, yb], axis=-1)

    return pl.pallas_call(
        body,
        grid=(32,),
        in_specs=[pl.BlockSpec((64, 200, 128), lambda it: (it, 0, 0))],
        out_specs=pl.BlockSpec((200, 64, 128), lambda it: (0, 0, it)),
        out_shape=jax.ShapeDtypeStruct((200, 64, 4096), jnp.float32),
    )(z)


def kernel(indices, pe):
    b0, b1 = indices.shape
    flat = indices.reshape(b0 * b1).astype(jnp.int32)
    z = _pair_gather(pe, flat)
    y = _retile(z)
    return jnp.transpose(y, (2, 0, 1))


# speedup vs baseline: 31.1465x; 1.0017x over previous
import functools

import jax
import jax.numpy as jnp
from jax import lax
from jax.experimental import pallas as pl
from jax.experimental.pallas import tpu as pltpu
from jax.experimental.pallas import tpu_sc as plsc

D_MODEL = 64

_NC = 2
_NS = 16
_NW = _NC * _NS
_NB = 2
_GL = 104


def _pair_gather(table, idx_flat):
    B = idx_flat.shape[0]
    b_per_w = B // _NW
    h_per_w = b_per_w // 2
    n_pairs = 64
    mesh = plsc.VectorSubcoreMesh(core_axis_name="c", subcore_axis_name="s")

    @functools.partial(
        pl.kernel,
        mesh=mesh,
        compiler_params=pltpu.CompilerParams(
            use_tc_tiling_on_sc=False, needs_layout_passes=False
        ),
        out_type=jax.ShapeDtypeStruct((B // 400, 200, 2 * D_MODEL),
                                      jnp.float32),
        scratch_types=[
            pltpu.VMEM((b_per_w,), jnp.int32),
            pltpu.VMEM((h_per_w,), jnp.int32),
            pltpu.VMEM((h_per_w,), jnp.int32),
            pltpu.VMEM((_NB, 4, _GL, D_MODEL), jnp.float32),
            pltpu.SemaphoreType.DMA((_NB,)),
            pltpu.SemaphoreType.DMA((_NB,)),
        ],
    )
    def k(table_hbm, idx_hbm, out_hbm,
          idx_v, idxe_v, idxo_v, rows_v, gsem, wsem):
        wid = lax.axis_index("s") * _NC + lax.axis_index("c")
        qbase = wid * n_pairs
        pltpu.sync_copy(idx_hbm.at[pl.ds(wid * b_per_w, b_per_w)], idx_v)

        lanes = lax.iota(jnp.int32, 16)

        def deint(g, carry):
            pos = lanes * 2 + g * 32
            idxe_v[pl.ds(g * 16, 16)] = plsc.load_gather(idx_v, [pos])
            idxo_v[pl.ds(g * 16, 16)] = plsc.load_gather(idx_v, [pos + 1])
            return carry

        lax.fori_loop(0, h_per_w // 16, deint, 0)

        def issue_gather(pi, b):
            off0 = pi * 200
            off1 = pi * 200 + 96
            for s, (iv, off) in enumerate(
                    ((idxe_v, off0), (idxe_v, off1),
                     (idxo_v, off0), (idxo_v, off1))):
                pltpu.async_copy(
                    table_hbm.at[iv.at[pl.ds(off, _GL)]],
                    rows_v.at[b, s],
                    gsem.at[b],
                )

        def wait_gather(b):
            for _ in range(4):
                pltpu.make_async_copy(
                    table_hbm.at[pl.ds(0, _GL)], rows_v.at[b, 0], gsem.at[b]
                ).wait()

        def issue_write(pi, b):
            il0 = 2 * pi
            par = il0 // 64
            q0 = qbase + il0 - 64 * par
            mb = 100 * par
            for s, (q, r0, c0) in enumerate(
                    ((q0, 0, 0), (q0 + 1, 4, 0),
                     (q0, 0, D_MODEL), (q0 + 1, 4, D_MODEL))):
                pltpu.async_copy(
                    rows_v.at[b, s, pl.ds(r0, 100)],
                    out_hbm.at[q, pl.ds(mb, 100), pl.ds(c0, D_MODEL)],
                    wsem.at[b],
                )

        def wait_write(b):
            for _ in range(4):
                pltpu.make_async_copy(
                    rows_v.at[b, 0, pl.ds(0, 100)],
                    out_hbm.at[qbase, pl.ds(0, 100), pl.ds(0, D_MODEL)],
                    wsem.at[b],
                ).wait()

        issue_gather(0, 0)

        def body(i, carry):
            b = lax.rem(i, _NB)
            wait_gather(b)
            issue_write(i, b)
            nxt = i + 1

            @pl.when(nxt < n_pairs)
            def _():
                bn = lax.rem(nxt, _NB)

                @pl.when(nxt >= _NB)
                def _():
                    wait_write(bn)

                issue_gather(nxt, bn)

            return carry

        lax.fori_loop(0, n_pairs, body, 0)

        for j in range(_NB):
            wait_write(j)

    return k(table, idx_flat)


def _retile(z):
    def body(z_ref, y_ref):
        zt = z_ref[...].reshape(64, 200 * 128).T
        ya = zt[:100 * 128].reshape(200, 64, 64)
        yb = zt[100 * 128:].reshape(200, 64, 64)
        y_ref[...] = jnp.concatenate([ya, yb], axis=-1)

    return pl.pallas_call(
        body,
        grid=(32,),
        in_specs=[pl.BlockSpec((64, 200, 128), lambda it: (it, 0, 0))],
        out_specs=pl.BlockSpec((200, 64, 128), lambda it: (0, 0, it)),
        out_shape=jax.ShapeDtypeStruct((200, 64, 4096), jnp.float32),
    )(z)


def kernel(indices, pe):
    b0, b1 = indices.shape
    flat = indices.reshape(b0 * b1).astype(jnp.int32)
    z = _pair_gather(pe, flat)
    y = _retile(z)
    return jnp.transpose(y, (2, 0, 1))
